# baseline (speedup 1.0000x reference)
import functools

import jax
import jax.numpy as jnp
from jax import lax
from jax.experimental import pallas as pl
from jax.experimental.pallas import tpu as pltpu
from jax.experimental.pallas import tpu_sc as plsc


_SCALE = 32.0

_NC = 2
_NS = 16
_NW = _NC * _NS

_B = 4
_SEQ = 8192
_D = 1024
_ROWS_W = _SEQ // _NW
_R = 16
_NCHUNK = _ROWS_W // _R
_VPR = _D // 16
_UNROLL = 8


def _fma_chunk(xbuf, pebuf):
    @plsc.parallel_loop(0, _R * _VPR, step=1, unroll=_UNROLL)
    def body(k):
        i = k >> 6
        sl = pl.ds((k & (_VPR - 1)) * 16, 16)
        xbuf[i, sl] = xbuf[i, sl] * _SCALE + pebuf[i, sl]


def _sc_body(x_hbm, pe_hbm, out_hbm, xa, xb, pea, peb, sla, slb, ssa, ssb, spe):
    wid = lax.axis_index("s") * _NC + lax.axis_index("c")
    row0 = wid * _ROWS_W
    xbufs = (xa, xb)
    lsems = (sla, slb)
    ssems = (ssa, ssb)
    pebufs = (pea, peb)

    pltpu.sync_copy(pe_hbm.at[pl.ds(row0, _R), :], pea)

    n_iter = _NCHUNK * _B
    loads = {}
    pe_loads = {}
    stores = {}
    loads[0] = pltpu.async_copy(
        x_hbm.at[0, pl.ds(row0, _R), :], xbufs[0], lsems[0])
    for t in range(n_iter):
        c, b = divmod(t, _B)
        cur = t % 2
        if t + 1 < n_iter:
            nxt = (t + 1) % 2
            if t - 1 >= 0:
                stores[t - 1].wait()
            c2, b2 = divmod(t + 1, _B)
            loads[t + 1] = pltpu.async_copy(
                x_hbm.at[b2, pl.ds(row0 + c2 * _R, _R), :],
                xbufs[nxt], lsems[nxt])
        if b == 0 and c + 1 < _NCHUNK:
            pe_loads[c + 1] = pltpu.async_copy(
                pe_hbm.at[pl.ds(row0 + (c + 1) * _R, _R), :],
                pebufs[(c + 1) % 2], spe)
        if b == 0 and c > 0:
            pe_loads[c].wait()
        loads[t].wait()
        _fma_chunk(xbufs[cur], pebufs[c % 2])
        stores[t] = pltpu.async_copy(
            xbufs[cur], out_hbm.at[b, pl.ds(row0 + c * _R, _R), :],
            ssems[cur])
    stores[n_iter - 2].wait()
    stores[n_iter - 1].wait()


def _sc_call(x, pos_emb):
    mesh = plsc.VectorSubcoreMesh(core_axis_name="c", subcore_axis_name="s")
    run = functools.partial(
        pl.kernel,
        mesh=mesh,
        out_type=jax.ShapeDtypeStruct((_B, _SEQ, _D), jnp.float32),
        scratch_types=[
            pltpu.VMEM((_R, _D), jnp.float32),
            pltpu.VMEM((_R, _D), jnp.float32),
            pltpu.VMEM((_R, _D), jnp.float32),
            pltpu.VMEM((_R, _D), jnp.float32),
            pltpu.SemaphoreType.DMA,
            pltpu.SemaphoreType.DMA,
            pltpu.SemaphoreType.DMA,
            pltpu.SemaphoreType.DMA,
            pltpu.SemaphoreType.DMA,
        ],
    )(_sc_body)
    return run(x, pos_emb)


def kernel(x, pos_emb):
    b, s, d = x.shape
    return _sc_call(x, pos_emb[:s])


# speedup vs baseline: 1.0252x; 1.0252x over previous
import functools

import jax
import jax.numpy as jnp
from jax import lax
from jax.experimental import pallas as pl
from jax.experimental.pallas import tpu as pltpu
from jax.experimental.pallas import tpu_sc as plsc


_SCALE = 32.0

_NC = 2
_NS = 16
_NW = _NC * _NS

_B = 4
_SEQ = 8192
_D = 1024
_ROWS_W = _SEQ // _NW
_R = 16
_NCHUNK = _ROWS_W // _R
_VPR = _D // 16
_UNROLL = 8


def _fma_chunk(xbuf, pebuf):
    @plsc.parallel_loop(0, _R * _VPR, step=1, unroll=_UNROLL)
    def body(k):
        i = k >> 6
        sl = pl.ds((k & (_VPR - 1)) * 16, 16)
        xbuf[i, sl] = xbuf[i, sl] * _SCALE + pebuf[i, sl]


_NBUF = 4


def _sc_body(x_hbm, pe_hbm, out_hbm,
             x0, x1, x2, x3, pea, peb,
             sl0, sl1, sl2, sl3, ss0, ss1, ss2, ss3, spe):
    wid = lax.axis_index("s") * _NC + lax.axis_index("c")
    row0 = wid * _ROWS_W
    xbufs = (x0, x1, x2, x3)
    lsems = (sl0, sl1, sl2, sl3)
    ssems = (ss0, ss1, ss2, ss3)
    pebufs = (pea, peb)

    pltpu.sync_copy(pe_hbm.at[pl.ds(row0, _R), :], pea)

    n_iter = _NCHUNK * _B
    loads = {}
    pe_loads = {}
    stores = {}
    for p in range(min(_NBUF - 1, n_iter)):
        cp, bp = divmod(p, _B)
        loads[p] = pltpu.async_copy(
            x_hbm.at[bp, pl.ds(row0 + cp * _R, _R), :],
            xbufs[p % _NBUF], lsems[p % _NBUF])
    for t in range(n_iter):
        c, b = divmod(t, _B)
        cur = t % _NBUF
        if t + _NBUF - 1 < n_iter:
            nxt = (t + _NBUF - 1) % _NBUF
            if t - 1 >= 0:
                stores[t - 1].wait()
            c2, b2 = divmod(t + _NBUF - 1, _B)
            loads[t + _NBUF - 1] = pltpu.async_copy(
                x_hbm.at[b2, pl.ds(row0 + c2 * _R, _R), :],
                xbufs[nxt], lsems[nxt])
        if b == 0 and c + 1 < _NCHUNK:
            pe_loads[c + 1] = pltpu.async_copy(
                pe_hbm.at[pl.ds(row0 + (c + 1) * _R, _R), :],
                pebufs[(c + 1) % 2], spe)
        if b == 0 and c > 0:
            pe_loads[c].wait()
        loads[t].wait()
        _fma_chunk(xbufs[cur], pebufs[c % 2])
        stores[t] = pltpu.async_copy(
            xbufs[cur], out_hbm.at[b, pl.ds(row0 + c * _R, _R), :],
            ssems[cur])
    for t in range(max(0, n_iter - _NBUF), n_iter):
        if t not in stores:
            continue
        stores[t].wait()


def _sc_call(x, pos_emb):
    mesh = plsc.VectorSubcoreMesh(core_axis_name="c", subcore_axis_name="s")
    run = functools.partial(
        pl.kernel,
        mesh=mesh,
        out_type=jax.ShapeDtypeStruct((_B, _SEQ, _D), jnp.float32),
        scratch_types=(
            [pltpu.VMEM((_R, _D), jnp.float32)] * (_NBUF + 2)
            + [pltpu.SemaphoreType.DMA] * (2 * _NBUF + 1)
        ),
    )(_sc_body)
    return run(x, pos_emb)


def kernel(x, pos_emb):
    b, s, d = x.shape
    return _sc_call(x, pos_emb[:s])


# speedup vs baseline: 1.1336x; 1.1057x over previous
import functools

import jax
import jax.numpy as jnp
from jax import lax
from jax.experimental import pallas as pl
from jax.experimental.pallas import tpu as pltpu
from jax.experimental.pallas import tpu_sc as plsc


_SCALE = 32.0

_NC = 2
_NS = 16
_NW = _NC * _NS

_B = 4
_SEQ = 8192
_D = 1024
_ROWS_W = _SEQ // _NW
_R = 8
_NCHUNK = _ROWS_W // _R
_VPR = _D // 16
_NSET = 3


def _fma_chunk4(x4, pebuf):
    @plsc.parallel_loop(0, _R * _VPR, step=1, unroll=4)
    def body(k):
        i = k >> 6
        sl = pl.ds((k & (_VPR - 1)) * 16, 16)
        pe = pebuf[i, sl]
        for xb in x4:
            xb[i, sl] = xb[i, sl] * _SCALE + pe


def _sc_body(x_hbm, pe_hbm, out_hbm, *scratch):
    xbufs = scratch[0:_NSET * _B]
    pebufs = scratch[_NSET * _B:_NSET * _B + 2]
    lsems = scratch[_NSET * _B + 2:_NSET * _B + 2 + _NSET]
    ssems = scratch[_NSET * _B + 2 + _NSET:_NSET * _B + 2 + 2 * _NSET]
    spe = scratch[-1]

    wid = lax.axis_index("s") * _NC + lax.axis_index("c")
    row0 = wid * _ROWS_W

    def xset(c):
        s = c % _NSET
        return xbufs[_B * s:_B * (s + 1)]

    def issue_loads(c):
        bs = xset(c)
        sem = lsems[c % _NSET]
        return [
            pltpu.async_copy(
                x_hbm.at[b, pl.ds(row0 + c * _R, _R), :], bs[b], sem)
            for b in range(_B)
        ]

    def issue_stores(c):
        bs = xset(c)
        sem = ssems[c % _NSET]
        return [
            pltpu.async_copy(
                bs[b], out_hbm.at[b, pl.ds(row0 + c * _R, _R), :], sem)
            for b in range(_B)
        ]

    pltpu.sync_copy(pe_hbm.at[pl.ds(row0, _R), :], pebufs[0])

    loads = {}
    pe_loads = {}
    stores = {}
    for c in range(min(_NSET - 1, _NCHUNK)):
        loads[c] = issue_loads(c)
    for c in range(_NCHUNK):
        if c + 1 < _NCHUNK:
            pe_loads[c + 1] = pltpu.async_copy(
                pe_hbm.at[pl.ds(row0 + (c + 1) * _R, _R), :],
                pebufs[(c + 1) % 2], spe)
        if c > 0:
            pe_loads[c].wait()
        for h in loads[c]:
            h.wait()
        _fma_chunk4(xset(c), pebufs[c % 2])
        stores[c] = issue_stores(c)
        if c + _NSET - 1 < _NCHUNK:
            if c - 1 >= 0:
                for h in stores[c - 1]:
                    h.wait()
            loads[c + _NSET - 1] = issue_loads(c + _NSET - 1)
    for c in range(max(0, _NCHUNK - _NSET), _NCHUNK):
        for h in stores[c]:
            h.wait()


def _sc_call(x, pos_emb):
    mesh = plsc.VectorSubcoreMesh(core_axis_name="c", subcore_axis_name="s")
    run = functools.partial(
        pl.kernel,
        mesh=mesh,
        out_type=jax.ShapeDtypeStruct((_B, _SEQ, _D), jnp.float32),
        scratch_types=(
            [pltpu.VMEM((_R, _D), jnp.float32)] * (_NSET * _B + 2)
            + [pltpu.SemaphoreType.DMA] * (2 * _NSET + 1)
        ),
    )(_sc_body)
    return run(x, pos_emb)


def kernel(x, pos_emb):
    b, s, d = x.shape
    return _sc_call(x, pos_emb[:s])
